# baseline (speedup 1.0000x reference)
import functools

import jax
import jax.numpy as jnp
from jax import lax
from jax.experimental import pallas as pl
from jax.experimental.pallas import tpu as pltpu
from jax.experimental.pallas import tpu_sc as plsc

B = 4096
F = 64
IMG_DIM = 512
NC, NS = 2, 16
NW = NC * NS
BPW = B // NW


def _sc_gather(user, item, user_factors, item_factors, user_biases, item_biases):
    mesh = plsc.VectorSubcoreMesh(
        core_axis_name="c", subcore_axis_name="s", num_cores=NC, num_subcores=NS
    )

    ub2 = user_biases.reshape(-1, 16)
    ib2 = item_biases.reshape(-1, 16)

    @functools.partial(
        pl.kernel,
        out_type=(
            jax.ShapeDtypeStruct((B, F), jnp.float32),
            jax.ShapeDtypeStruct((B, F), jnp.float32),
            jax.ShapeDtypeStruct((B,), jnp.float32),
            jax.ShapeDtypeStruct((B,), jnp.float32),
        ),
        mesh=mesh,
        scratch_types=[
            pltpu.VMEM((BPW,), jnp.int32),
            pltpu.VMEM((BPW,), jnp.int32),
            pltpu.VMEM((BPW,), jnp.int32),
            pltpu.VMEM((BPW,), jnp.int32),
            pltpu.VMEM((BPW, F), jnp.float32),
            pltpu.VMEM((BPW, F), jnp.float32),
            pltpu.VMEM((BPW, 16), jnp.float32),
            pltpu.VMEM((BPW, 16), jnp.float32),
            pltpu.VMEM((BPW,), jnp.float32),
            pltpu.VMEM((BPW,), jnp.float32),
            pltpu.SemaphoreType.DMA,
        ],
        compiler_params=pltpu.CompilerParams(
            use_tc_tiling_on_sc=False, needs_layout_passes=False),
    )
    def k(user_hbm, item_hbm, uf_hbm, if_hbm, ub_hbm, ib_hbm,
          uf_out, if_out, ub_out, ib_out,
          uidx_v, iidx_v, ubrow_v, ibrow_v, uf_v, if_v, ubr_v, ibr_v,
          ubsel_v, ibsel_v, sem):
        wid = lax.axis_index("s") * NC + lax.axis_index("c")
        base = wid * BPW
        pltpu.sync_copy(user_hbm.at[pl.ds(base, BPW)], uidx_v)
        pltpu.sync_copy(item_hbm.at[pl.ds(base, BPW)], iidx_v)
        for g in range(BPW // 16):
            sl = pl.ds(g * 16, 16)
            ubrow_v[sl] = lax.shift_right_logical(uidx_v[sl], 4)
            ibrow_v[sl] = lax.shift_right_logical(iidx_v[sl], 4)
        c1 = pltpu.async_copy(uf_hbm.at[uidx_v], uf_v, sem)
        c2 = pltpu.async_copy(if_hbm.at[iidx_v], if_v, sem)
        c3 = pltpu.async_copy(ub_hbm.at[ubrow_v], ubr_v, sem)
        c4 = pltpu.async_copy(ib_hbm.at[ibrow_v], ibr_v, sem)
        c1.wait()
        c2.wait()
        c3.wait()
        c4.wait()
        for g in range(BPW // 16):
            sl = pl.ds(g * 16, 16)
            rows = g * 16 + lax.iota(jnp.int32, 16)
            ubsel_v[sl] = plsc.load_gather(ubr_v, [rows, uidx_v[sl] & 15])
            ibsel_v[sl] = plsc.load_gather(ibr_v, [rows, iidx_v[sl] & 15])
        pltpu.sync_copy(uf_v, uf_out.at[pl.ds(base, BPW)])
        pltpu.sync_copy(if_v, if_out.at[pl.ds(base, BPW)])
        pltpu.sync_copy(ubsel_v, ub_out.at[pl.ds(base, BPW)])
        pltpu.sync_copy(ibsel_v, ib_out.at[pl.ds(base, BPW)])

    return k(user, item, user_factors, item_factors, ub2, ib2)


def _tc_combine(image, W_img, b_img, uf, itf, ub2, ib2):
    BLK = 512
    NBLK = B // BLK

    def body(img_ref, w_ref, b_ref, uf_ref, itf_ref, ub_ref, ib_ref, out_ref):
        imf = jnp.dot(img_ref[...], w_ref[...],
                      preferred_element_type=jnp.float32) + b_ref[...]
        itv = 0.5 * (imf + itf_ref[...])
        pred = ub_ref[0, 0, :] + ib_ref[0, 0, :] + jnp.sum(uf_ref[...] * itv, axis=1)
        out_ref[0, 0, :] = pred

    out = pl.pallas_call(
        body,
        grid=(NBLK,),
        in_specs=[
            pl.BlockSpec((BLK, IMG_DIM), lambda i: (i, 0)),
            pl.BlockSpec((IMG_DIM, F), lambda i: (0, 0)),
            pl.BlockSpec((1, F), lambda i: (0, 0)),
            pl.BlockSpec((BLK, F), lambda i: (i, 0)),
            pl.BlockSpec((BLK, F), lambda i: (i, 0)),
            pl.BlockSpec((1, 1, BLK), lambda i: (i, 0, 0)),
            pl.BlockSpec((1, 1, BLK), lambda i: (i, 0, 0)),
        ],
        out_specs=pl.BlockSpec((1, 1, BLK), lambda i: (i, 0, 0)),
        out_shape=jax.ShapeDtypeStruct((NBLK, 1, BLK), jnp.float32),
    )(image, W_img, b_img.reshape(1, F), uf, itf, ub2, ib2)
    return out.reshape(B)


def kernel(image, user, item, user_factors, item_factors, user_biases,
           item_biases, W_img, b_img):
    uf, itf, ub, ib = _sc_gather(
        user.astype(jnp.int32), item.astype(jnp.int32),
        user_factors, item_factors, user_biases, item_biases)
    ub2 = ub.reshape(B // 512, 1, 512)
    ib2 = ib.reshape(B // 512, 1, 512)
    return _tc_combine(image, W_img, b_img, uf, itf, ub2, ib2)


# speedup vs baseline: 1.0373x; 1.0373x over previous
import functools

import jax
import jax.numpy as jnp
from jax import lax
from jax.experimental import pallas as pl
from jax.experimental.pallas import tpu as pltpu
from jax.experimental.pallas import tpu_sc as plsc

B = 4096
F = 64
IMG_DIM = 512
NC, NS = 2, 16
NW = NC * NS
BPW = B // NW


def _sc_gather(user, item, user_factors, item_factors, user_biases, item_biases):
    mesh = plsc.VectorSubcoreMesh(
        core_axis_name="c", subcore_axis_name="s", num_cores=NC, num_subcores=NS
    )

    @functools.partial(
        pl.kernel,
        out_type=(
            jax.ShapeDtypeStruct((B, F), jnp.float32),
            jax.ShapeDtypeStruct((B, F), jnp.float32),
            jax.ShapeDtypeStruct((B, 1), jnp.float32),
            jax.ShapeDtypeStruct((B, 1), jnp.float32),
        ),
        mesh=mesh,
        scratch_types=[
            pltpu.VMEM((BPW,), jnp.int32),
            pltpu.VMEM((BPW,), jnp.int32),
            pltpu.VMEM((BPW, F), jnp.float32),
            pltpu.VMEM((BPW, F), jnp.float32),
            pltpu.VMEM((BPW, 1), jnp.float32),
            pltpu.VMEM((BPW, 1), jnp.float32),
            pltpu.SemaphoreType.DMA,
        ],
        compiler_params=pltpu.CompilerParams(needs_layout_passes=False),
    )
    def k(user_hbm, item_hbm, uf_hbm, if_hbm, ub_hbm, ib_hbm,
          uf_out, if_out, ub_out, ib_out,
          uidx_v, iidx_v, uf_v, if_v, ub_v, ib_v, sem):
        wid = lax.axis_index("s") * NC + lax.axis_index("c")
        base = wid * BPW
        pltpu.sync_copy(user_hbm.at[pl.ds(base, BPW)], uidx_v)
        pltpu.sync_copy(item_hbm.at[pl.ds(base, BPW)], iidx_v)
        lane = lax.iota(jnp.int32, 16)
        for g in range(BPW // 16):
            uchunk = uidx_v[pl.ds(g * 16, 16)]
            ichunk = iidx_v[pl.ds(g * 16, 16)]
            for l in range(16):
                j = g * 16 + l
                su = jnp.sum(jnp.where(lane == l, uchunk, 0))
                si = jnp.sum(jnp.where(lane == l, ichunk, 0))
                pltpu.async_copy(
                    uf_hbm.at[pl.ds(su, 1)], uf_v.at[pl.ds(j, 1)], sem)
                pltpu.async_copy(
                    if_hbm.at[pl.ds(si, 1)], if_v.at[pl.ds(j, 1)], sem)
                pltpu.async_copy(
                    ub_hbm.at[pl.ds(su, 1)], ub_v.at[pl.ds(j, 1)], sem)
                pltpu.async_copy(
                    ib_hbm.at[pl.ds(si, 1)], ib_v.at[pl.ds(j, 1)], sem)
        pltpu.make_async_copy(uf_hbm.at[pl.ds(0, BPW)], uf_v, sem).wait()
        pltpu.make_async_copy(if_hbm.at[pl.ds(0, BPW)], if_v, sem).wait()
        pltpu.make_async_copy(ub_hbm.at[pl.ds(0, BPW)], ub_v, sem).wait()
        pltpu.make_async_copy(ib_hbm.at[pl.ds(0, BPW)], ib_v, sem).wait()
        pltpu.sync_copy(uf_v, uf_out.at[pl.ds(base, BPW)])
        pltpu.sync_copy(if_v, if_out.at[pl.ds(base, BPW)])
        pltpu.sync_copy(ub_v, ub_out.at[pl.ds(base, BPW)])
        pltpu.sync_copy(ib_v, ib_out.at[pl.ds(base, BPW)])

    return k(user, item, user_factors, item_factors, user_biases, item_biases)


def _tc_combine(image, W_img, b_img, uf, itf, ub2, ib2):
    BLK = 512
    NBLK = B // BLK

    def body(img_ref, w_ref, b_ref, uf_ref, itf_ref, ub_ref, ib_ref, out_ref):
        imf = jnp.dot(img_ref[...], w_ref[...],
                      preferred_element_type=jnp.float32) + b_ref[...]
        itv = 0.5 * (imf + itf_ref[...])
        pred = ub_ref[0, 0, :] + ib_ref[0, 0, :] + jnp.sum(uf_ref[...] * itv, axis=1)
        out_ref[0, 0, :] = pred

    out = pl.pallas_call(
        body,
        grid=(NBLK,),
        in_specs=[
            pl.BlockSpec((BLK, IMG_DIM), lambda i: (i, 0)),
            pl.BlockSpec((IMG_DIM, F), lambda i: (0, 0)),
            pl.BlockSpec((1, F), lambda i: (0, 0)),
            pl.BlockSpec((BLK, F), lambda i: (i, 0)),
            pl.BlockSpec((BLK, F), lambda i: (i, 0)),
            pl.BlockSpec((1, 1, BLK), lambda i: (i, 0, 0)),
            pl.BlockSpec((1, 1, BLK), lambda i: (i, 0, 0)),
        ],
        out_specs=pl.BlockSpec((1, 1, BLK), lambda i: (i, 0, 0)),
        out_shape=jax.ShapeDtypeStruct((NBLK, 1, BLK), jnp.float32),
    )(image, W_img, b_img.reshape(1, F), uf, itf, ub2, ib2)
    return out.reshape(B)


def kernel(image, user, item, user_factors, item_factors, user_biases,
           item_biases, W_img, b_img):
    uf, itf, ub, ib = _sc_gather(
        user.astype(jnp.int32), item.astype(jnp.int32),
        user_factors, item_factors, user_biases, item_biases)
    ub2 = ub.reshape(B // 512, 1, 512)
    ib2 = ib.reshape(B // 512, 1, 512)
    return _tc_combine(image, W_img, b_img, uf, itf, ub2, ib2)


# speedup vs baseline: 3.6177x; 3.4875x over previous
import functools

import jax
import jax.numpy as jnp
from jax import lax
from jax.experimental import pallas as pl
from jax.experimental.pallas import tpu as pltpu
from jax.experimental.pallas import tpu_sc as plsc

B = 4096
F = 64
IMG_DIM = 512
NC, NS = 2, 16
NW = NC * NS
BPW = B // NW
NBUF = 8


def _sc_user_gather(user, uf_t):
    mesh = plsc.VectorSubcoreMesh(
        core_axis_name="c", subcore_axis_name="s", num_cores=NC, num_subcores=NS
    )

    @functools.partial(
        pl.kernel,
        out_type=jax.ShapeDtypeStruct((F, B), jnp.float32),
        mesh=mesh,
        scratch_types=[
            pltpu.VMEM((BPW,), jnp.int32),
            pltpu.VMEM((NBUF, F, 128), jnp.float32),
            pltpu.VMEM((F, BPW), jnp.float32),
        ] + [pltpu.SemaphoreType.DMA] * NBUF,
        compiler_params=pltpu.CompilerParams(needs_layout_passes=False),
    )
    def k(user_hbm, uft_hbm, uft_out, uidx_v, blocks_v, uf_vt, *sems):
        wid = lax.axis_index("s") * NC + lax.axis_index("c")
        base = wid * BPW
        pltpu.sync_copy(user_hbm.at[pl.ds(base, BPW)], uidx_v)
        lane = lax.iota(jnp.int32, 16)

        def idx_at(j):
            chunk = uidx_v[pl.ds((j // 16) * 16, 16)]
            return jnp.sum(jnp.where(lane == j % 16, chunk, 0))

        def start_fetch(j):
            su = idx_at(j)
            t0 = pl.multiple_of((su >> 7) << 7, 128)
            pltpu.async_copy(
                uft_hbm.at[:, pl.ds(t0, 128)],
                blocks_v.at[j % NBUF], sems[j % NBUF])

        for j in range(NBUF):
            start_fetch(j)
        for j in range(BPW):
            su = idx_at(j)
            c = su & 127
            pltpu.make_async_copy(
                uft_hbm.at[:, pl.ds(0, 128)],
                blocks_v.at[j % NBUF], sems[j % NBUF]).wait()
            for g in range(F // 16):
                rows = g * 16 + lane
                vals = plsc.load_gather(
                    blocks_v.at[j % NBUF], [rows, jnp.broadcast_to(c, (16,))])
                plsc.store_scatter(
                    uf_vt, [rows, jnp.broadcast_to(jnp.int32(j), (16,))], vals)
            if j + NBUF < BPW:
                start_fetch(j + NBUF)
        pltpu.sync_copy(uf_vt, uft_out.at[:, pl.ds(base, BPW)])

    return k(user, uf_t)


def _sc_item_gather(user, item, item_factors, ub16, ib16):
    mesh = plsc.VectorSubcoreMesh(
        core_axis_name="c", subcore_axis_name="s", num_cores=NC, num_subcores=NS
    )

    @functools.partial(
        pl.kernel,
        out_type=(
            jax.ShapeDtypeStruct((B, F), jnp.float32),
            jax.ShapeDtypeStruct((B,), jnp.float32),
            jax.ShapeDtypeStruct((B,), jnp.float32),
        ),
        mesh=mesh,
        scratch_types=[
            pltpu.VMEM((BPW,), jnp.int32),
            pltpu.VMEM((BPW,), jnp.int32),
            pltpu.VMEM((BPW,), jnp.int32),
            pltpu.VMEM((BPW,), jnp.int32),
            pltpu.VMEM((BPW, F), jnp.float32),
            pltpu.VMEM((BPW, 16), jnp.float32),
            pltpu.VMEM((BPW, 16), jnp.float32),
            pltpu.VMEM((BPW,), jnp.float32),
            pltpu.VMEM((BPW,), jnp.float32),
            pltpu.SemaphoreType.DMA,
        ],
        compiler_params=pltpu.CompilerParams(
            use_tc_tiling_on_sc=False, needs_layout_passes=False),
    )
    def k(user_hbm, item_hbm, if_hbm, ub_hbm, ib_hbm,
          if_out, ub_out, ib_out,
          uidx_v, iidx_v, ubrow_v, ibrow_v, if_v, ubr_v, ibr_v,
          ubsel_v, ibsel_v, sem):
        wid = lax.axis_index("s") * NC + lax.axis_index("c")
        base = wid * BPW
        pltpu.sync_copy(user_hbm.at[pl.ds(base, BPW)], uidx_v)
        pltpu.sync_copy(item_hbm.at[pl.ds(base, BPW)], iidx_v)
        for g in range(BPW // 16):
            sl = pl.ds(g * 16, 16)
            ubrow_v[sl] = lax.shift_right_logical(uidx_v[sl], 4)
            ibrow_v[sl] = lax.shift_right_logical(iidx_v[sl], 4)
        c1 = pltpu.async_copy(if_hbm.at[iidx_v], if_v, sem)
        c2 = pltpu.async_copy(ub_hbm.at[ubrow_v], ubr_v, sem)
        c3 = pltpu.async_copy(ib_hbm.at[ibrow_v], ibr_v, sem)
        c1.wait()
        c2.wait()
        c3.wait()
        lane = lax.iota(jnp.int32, 16)
        for g in range(BPW // 16):
            sl = pl.ds(g * 16, 16)
            rows = g * 16 + lane
            ubsel_v[sl] = plsc.load_gather(ubr_v, [rows, uidx_v[sl] & 15])
            ibsel_v[sl] = plsc.load_gather(ibr_v, [rows, iidx_v[sl] & 15])
        pltpu.sync_copy(if_v, if_out.at[pl.ds(base, BPW)])
        pltpu.sync_copy(ubsel_v, ub_out.at[pl.ds(base, BPW)])
        pltpu.sync_copy(ibsel_v, ib_out.at[pl.ds(base, BPW)])

    return k(user, item, item_factors, ub16, ib16)


def _tc_combine(image_t, W_img_t, b_img, uf_t, itf, ub3, ib3):
    BLK = 512
    NBLK = B // BLK

    def body(imgt_ref, wt_ref, b_ref, uft_ref, itf_ref, ub_ref, ib_ref, out_ref):
        imf_t = jnp.dot(wt_ref[...], imgt_ref[...],
                        preferred_element_type=jnp.float32) + b_ref[...].T
        itv_t = 0.5 * (imf_t + itf_ref[...].T)
        pred = ub_ref[0, 0, :] + ib_ref[0, 0, :] + jnp.sum(
            uft_ref[...] * itv_t, axis=0)
        out_ref[0, 0, :] = pred

    out = pl.pallas_call(
        body,
        grid=(NBLK,),
        in_specs=[
            pl.BlockSpec((IMG_DIM, BLK), lambda i: (0, i)),
            pl.BlockSpec((F, IMG_DIM), lambda i: (0, 0)),
            pl.BlockSpec((1, F), lambda i: (0, 0)),
            pl.BlockSpec((F, BLK), lambda i: (0, i)),
            pl.BlockSpec((BLK, F), lambda i: (i, 0)),
            pl.BlockSpec((1, 1, BLK), lambda i: (i, 0, 0)),
            pl.BlockSpec((1, 1, BLK), lambda i: (i, 0, 0)),
        ],
        out_specs=pl.BlockSpec((1, 1, BLK), lambda i: (i, 0, 0)),
        out_shape=jax.ShapeDtypeStruct((NBLK, 1, BLK), jnp.float32),
    )(image_t, W_img_t, b_img.reshape(1, F), uf_t, itf, ub3, ib3)
    return out.reshape(B)


def kernel(image, user, item, user_factors, item_factors, user_biases,
           item_biases, W_img, b_img):
    user = user.astype(jnp.int32)
    item = item.astype(jnp.int32)
    uf_t = _sc_user_gather(user, user_factors.T)
    itf, ub, ib = _sc_item_gather(
        user, item, item_factors,
        user_biases.reshape(-1, 16), item_biases.reshape(-1, 16))
    ub3 = ub.reshape(B // 512, 1, 512)
    ib3 = ib.reshape(B // 512, 1, 512)
    return _tc_combine(image.T, W_img.T, b_img, uf_t, itf, ub3, ib3)


# speedup vs baseline: 4.2325x; 1.1700x over previous
import functools

import jax
import jax.numpy as jnp
from jax import lax
from jax.experimental import pallas as pl
from jax.experimental.pallas import tpu as pltpu
from jax.experimental.pallas import tpu_sc as plsc

B = 4096
F = 64
IMG_DIM = 512
NC, NS = 2, 16
NW = NC * NS
BPW = B // NW
NBUF = 8


def _sc_user_gather(user, item, uf_t, ub_t, ib_t):
    mesh = plsc.VectorSubcoreMesh(
        core_axis_name="c", subcore_axis_name="s", num_cores=NC, num_subcores=NS
    )

    @functools.partial(
        pl.kernel,
        out_type=(
            jax.ShapeDtypeStruct((B, F), jnp.float32),
            jax.ShapeDtypeStruct((B,), jnp.float32),
            jax.ShapeDtypeStruct((B,), jnp.float32),
        ),
        mesh=mesh,
        scratch_types=[
            pltpu.VMEM((BPW,), jnp.int32),
            pltpu.VMEM((BPW,), jnp.int32),
            pltpu.VMEM((NBUF, F, 128), jnp.float32),
            pltpu.VMEM((BPW, F), jnp.float32),
            pltpu.VMEM((BPW, 128), jnp.float32),
            pltpu.VMEM((BPW, 128), jnp.float32),
            pltpu.VMEM((BPW,), jnp.float32),
            pltpu.VMEM((BPW,), jnp.float32),
        ] + [pltpu.SemaphoreType.DMA] * NBUF,
        compiler_params=pltpu.CompilerParams(needs_layout_passes=False),
    )
    def k(user_hbm, item_hbm, uft_hbm, ubt_hbm, ibt_hbm,
          uf_out, ub_out, ib_out,
          uidx_v, iidx_v, blocks_v, uf_v, ubw_v, ibw_v, ubsel_v, ibsel_v,
          *sems):
        wid = lax.axis_index("s") * NC + lax.axis_index("c")
        base = wid * BPW
        pltpu.sync_copy(user_hbm.at[pl.ds(base, BPW)], uidx_v)
        pltpu.sync_copy(item_hbm.at[pl.ds(base, BPW)], iidx_v)
        lane = lax.iota(jnp.int32, 16)

        def idx_at(ref, j):
            chunk = ref[pl.ds((j // 16) * 16, 16)]
            return jnp.sum(jnp.where(lane == j % 16, chunk, 0))

        def start_fetch(j):
            su = idx_at(uidx_v, j)
            si = idx_at(iidx_v, j)
            t0u = pl.multiple_of((su >> 7) << 7, 128)
            t0i = pl.multiple_of((si >> 7) << 7, 128)
            sem = sems[j % NBUF]
            pltpu.async_copy(
                uft_hbm.at[:, pl.ds(t0u, 128)], blocks_v.at[j % NBUF], sem)
            pltpu.async_copy(
                ubt_hbm.at[:, pl.ds(t0u, 128)], ubw_v.at[pl.ds(j, 1)], sem)
            pltpu.async_copy(
                ibt_hbm.at[:, pl.ds(t0i, 128)], ibw_v.at[pl.ds(j, 1)], sem)

        def wait_fetch(j):
            sem = sems[j % NBUF]
            pltpu.make_async_copy(
                uft_hbm.at[:, pl.ds(0, 128)], blocks_v.at[j % NBUF], sem).wait()
            pltpu.make_async_copy(
                ubt_hbm.at[:, pl.ds(0, 128)], ubw_v.at[pl.ds(j, 1)], sem).wait()
            pltpu.make_async_copy(
                ibt_hbm.at[:, pl.ds(0, 128)], ibw_v.at[pl.ds(j, 1)], sem).wait()

        for j in range(NBUF):
            start_fetch(j)
        for g in range(BPW // 16):
            uchunk = uidx_v[pl.ds(g * 16, 16)]
            for l in range(16):
                j = g * 16 + l
                c = jnp.sum(jnp.where(lane == l, uchunk, 0)) & 127
                wait_fetch(j)
                for gg in range(F // 16):
                    rows = gg * 16 + lane
                    vals = plsc.load_gather(
                        blocks_v.at[j % NBUF],
                        [rows, jnp.broadcast_to(c, (16,))])
                    uf_v[j, pl.ds(gg * 16, 16)] = vals
                if j + NBUF < BPW:
                    start_fetch(j + NBUF)
            rows = g * 16 + lane
            sl = pl.ds(g * 16, 16)
            ubsel_v[sl] = plsc.load_gather(ubw_v, [rows, uchunk & 127])
            ibsel_v[sl] = plsc.load_gather(
                ibw_v, [rows, iidx_v[sl] & 127])
        pltpu.sync_copy(uf_v, uf_out.at[pl.ds(base, BPW)])
        pltpu.sync_copy(ubsel_v, ub_out.at[pl.ds(base, BPW)])
        pltpu.sync_copy(ibsel_v, ib_out.at[pl.ds(base, BPW)])

    return k(user, item, uf_t, ub_t, ib_t)


def _sc_item_gather(item, item_factors):
    mesh = plsc.VectorSubcoreMesh(
        core_axis_name="c", subcore_axis_name="s", num_cores=NC, num_subcores=NS
    )

    @functools.partial(
        pl.kernel,
        out_type=jax.ShapeDtypeStruct((B, F), jnp.float32),
        mesh=mesh,
        scratch_types=[
            pltpu.VMEM((BPW,), jnp.int32),
            pltpu.VMEM((BPW, F), jnp.float32),
            pltpu.SemaphoreType.DMA,
        ],
        compiler_params=pltpu.CompilerParams(
            use_tc_tiling_on_sc=False, needs_layout_passes=False),
    )
    def k(item_hbm, if_hbm, if_out, iidx_v, if_v, sem):
        wid = lax.axis_index("s") * NC + lax.axis_index("c")
        base = wid * BPW
        pltpu.sync_copy(item_hbm.at[pl.ds(base, BPW)], iidx_v)
        pltpu.async_copy(if_hbm.at[iidx_v], if_v, sem).wait()
        pltpu.sync_copy(if_v, if_out.at[pl.ds(base, BPW)])

    return k(item, item_factors)


def _tc_combine(image, W_img, b_img, uf, itf, ub3, ib3):
    BLK = 512
    NBLK = B // BLK

    def body(img_ref, w_ref, b_ref, uf_ref, itf_ref, ub_ref, ib_ref, out_ref):
        imf = jnp.dot(img_ref[...], w_ref[...],
                      preferred_element_type=jnp.float32) + b_ref[...]
        itv = 0.5 * (imf + itf_ref[...])
        pred = ub_ref[0, 0, :] + ib_ref[0, 0, :] + jnp.sum(uf_ref[...] * itv, axis=1)
        out_ref[0, 0, :] = pred

    out = pl.pallas_call(
        body,
        grid=(NBLK,),
        in_specs=[
            pl.BlockSpec((BLK, IMG_DIM), lambda i: (i, 0)),
            pl.BlockSpec((IMG_DIM, F), lambda i: (0, 0)),
            pl.BlockSpec((1, F), lambda i: (0, 0)),
            pl.BlockSpec((BLK, F), lambda i: (i, 0)),
            pl.BlockSpec((BLK, F), lambda i: (i, 0)),
            pl.BlockSpec((1, 1, BLK), lambda i: (i, 0, 0)),
            pl.BlockSpec((1, 1, BLK), lambda i: (i, 0, 0)),
        ],
        out_specs=pl.BlockSpec((1, 1, BLK), lambda i: (i, 0, 0)),
        out_shape=jax.ShapeDtypeStruct((NBLK, 1, BLK), jnp.float32),
    )(image, W_img, b_img.reshape(1, F), uf, itf, ub3, ib3)
    return out.reshape(B)


def kernel(image, user, item, user_factors, item_factors, user_biases,
           item_biases, W_img, b_img):
    user = user.astype(jnp.int32)
    item = item.astype(jnp.int32)
    uf, ub, ib = _sc_user_gather(
        user, item, user_factors.T, user_biases.T, item_biases.T)
    itf = _sc_item_gather(item, item_factors)
    ub3 = ub.reshape(B // 512, 1, 512)
    ib3 = ib.reshape(B // 512, 1, 512)
    return _tc_combine(image, W_img, b_img, uf, itf, ub3, ib3)


# speedup vs baseline: 4.5982x; 1.0864x over previous
import functools

import jax
import jax.numpy as jnp
from jax import lax
from jax.experimental import pallas as pl
from jax.experimental.pallas import tpu as pltpu
from jax.experimental.pallas import tpu_sc as plsc

B = 4096
F = 64
IMG_DIM = 512
NC, NS = 2, 16
NW = NC * NS
BPW = B // NW
NBUF = 4


def _sc_gather(user, item, uf_t, if_t, ub_t, ib_t):
    mesh = plsc.VectorSubcoreMesh(
        core_axis_name="c", subcore_axis_name="s", num_cores=NC, num_subcores=NS
    )

    @functools.partial(
        pl.kernel,
        out_type=(
            jax.ShapeDtypeStruct((B, F), jnp.float32),
            jax.ShapeDtypeStruct((B, F), jnp.float32),
            jax.ShapeDtypeStruct((B,), jnp.float32),
            jax.ShapeDtypeStruct((B,), jnp.float32),
        ),
        mesh=mesh,
        scratch_types=[
            pltpu.VMEM((BPW,), jnp.int32),
            pltpu.VMEM((BPW,), jnp.int32),
            pltpu.VMEM((NBUF, F, 128), jnp.float32),
            pltpu.VMEM((NBUF, F, 128), jnp.float32),
            pltpu.VMEM((BPW, F), jnp.float32),
            pltpu.VMEM((BPW, F), jnp.float32),
            pltpu.VMEM((NBUF, 128), jnp.float32),
            pltpu.VMEM((NBUF, 128), jnp.float32),
            pltpu.VMEM((BPW,), jnp.float32),
            pltpu.VMEM((BPW,), jnp.float32),
        ] + [pltpu.SemaphoreType.DMA] * NBUF,
        compiler_params=pltpu.CompilerParams(needs_layout_passes=False),
    )
    def k(user_hbm, item_hbm, uft_hbm, ift_hbm, ubt_hbm, ibt_hbm,
          uf_out, if_out, ub_out, ib_out,
          uidx_v, iidx_v, ublocks_v, iblocks_v, uf_v, if_v, ubw_v, ibw_v,
          ubsel_v, ibsel_v, *sems):
        wid = lax.axis_index("s") * NC + lax.axis_index("c")
        base = wid * BPW
        pltpu.sync_copy(user_hbm.at[pl.ds(base, BPW)], uidx_v)
        pltpu.sync_copy(item_hbm.at[pl.ds(base, BPW)], iidx_v)
        lane = lax.iota(jnp.int32, 16)

        def idx_at(ref, j):
            chunk = ref[pl.ds((j // 16) * 16, 16)]
            return jnp.sum(jnp.where(lane == j % 16, chunk, 0))

        def start_fetch(j):
            su = idx_at(uidx_v, j)
            si = idx_at(iidx_v, j)
            t0u = pl.multiple_of((su >> 7) << 7, 128)
            t0i = pl.multiple_of((si >> 7) << 7, 128)
            sem = sems[j % NBUF]
            pltpu.async_copy(
                uft_hbm.at[:, pl.ds(t0u, 128)], ublocks_v.at[j % NBUF], sem)
            pltpu.async_copy(
                ift_hbm.at[:, pl.ds(t0i, 128)], iblocks_v.at[j % NBUF], sem)
            pltpu.async_copy(
                ubt_hbm.at[:, pl.ds(t0u, 128)], ubw_v.at[pl.ds(j % NBUF, 1)], sem)
            pltpu.async_copy(
                ibt_hbm.at[:, pl.ds(t0i, 128)], ibw_v.at[pl.ds(j % NBUF, 1)], sem)

        def wait_fetch(j):
            sem = sems[j % NBUF]
            pltpu.make_async_copy(
                uft_hbm.at[:, pl.ds(0, 128)], ublocks_v.at[j % NBUF], sem).wait()
            pltpu.make_async_copy(
                ift_hbm.at[:, pl.ds(0, 128)], iblocks_v.at[j % NBUF], sem).wait()
            pltpu.make_async_copy(
                ubt_hbm.at[:, pl.ds(0, 128)], ubw_v.at[pl.ds(j % NBUF, 1)], sem).wait()
            pltpu.make_async_copy(
                ibt_hbm.at[:, pl.ds(0, 128)], ibw_v.at[pl.ds(j % NBUF, 1)], sem).wait()

        for j in range(NBUF):
            start_fetch(j)
        for g in range(BPW // 16):
            uchunk = uidx_v[pl.ds(g * 16, 16)]
            ichunk = iidx_v[pl.ds(g * 16, 16)]
            ubacc = jnp.zeros((16,), jnp.float32)
            ibacc = jnp.zeros((16,), jnp.float32)
            for l in range(16):
                j = g * 16 + l
                cu = jnp.sum(jnp.where(lane == l, uchunk, 0)) & 127
                ci = jnp.sum(jnp.where(lane == l, ichunk, 0)) & 127
                wait_fetch(j)
                for gg in range(F // 16):
                    rows = gg * 16 + lane
                    uvals = plsc.load_gather(
                        ublocks_v.at[j % NBUF],
                        [rows, jnp.broadcast_to(cu, (16,))])
                    uf_v[j, pl.ds(gg * 16, 16)] = uvals
                    ivals = plsc.load_gather(
                        iblocks_v.at[j % NBUF],
                        [rows, jnp.broadcast_to(ci, (16,))])
                    if_v[j, pl.ds(gg * 16, 16)] = ivals
                ubval = plsc.load_gather(
                    ubw_v, [jnp.broadcast_to(jnp.int32(j % NBUF), (16,)),
                            jnp.broadcast_to(cu, (16,))])
                ibval = plsc.load_gather(
                    ibw_v, [jnp.broadcast_to(jnp.int32(j % NBUF), (16,)),
                            jnp.broadcast_to(ci, (16,))])
                ubacc = jnp.where(lane == l, ubval, ubacc)
                ibacc = jnp.where(lane == l, ibval, ibacc)
                if j + NBUF < BPW:
                    start_fetch(j + NBUF)
            sl = pl.ds(g * 16, 16)
            ubsel_v[sl] = ubacc
            ibsel_v[sl] = ibacc
        pltpu.sync_copy(uf_v, uf_out.at[pl.ds(base, BPW)])
        pltpu.sync_copy(if_v, if_out.at[pl.ds(base, BPW)])
        pltpu.sync_copy(ubsel_v, ub_out.at[pl.ds(base, BPW)])
        pltpu.sync_copy(ibsel_v, ib_out.at[pl.ds(base, BPW)])

    return k(user, item, uf_t, if_t, ub_t, ib_t)


def _tc_combine(image, W_img, b_img, uf, itf, ub3, ib3):
    BLK = 512
    NBLK = B // BLK

    def body(img_ref, w_ref, b_ref, uf_ref, itf_ref, ub_ref, ib_ref, out_ref):
        imf = jnp.dot(img_ref[...], w_ref[...],
                      preferred_element_type=jnp.float32) + b_ref[...]
        itv = 0.5 * (imf + itf_ref[...])
        pred = ub_ref[0, 0, :] + ib_ref[0, 0, :] + jnp.sum(uf_ref[...] * itv, axis=1)
        out_ref[0, 0, :] = pred

    out = pl.pallas_call(
        body,
        grid=(NBLK,),
        in_specs=[
            pl.BlockSpec((BLK, IMG_DIM), lambda i: (i, 0)),
            pl.BlockSpec((IMG_DIM, F), lambda i: (0, 0)),
            pl.BlockSpec((1, F), lambda i: (0, 0)),
            pl.BlockSpec((BLK, F), lambda i: (i, 0)),
            pl.BlockSpec((BLK, F), lambda i: (i, 0)),
            pl.BlockSpec((1, 1, BLK), lambda i: (i, 0, 0)),
            pl.BlockSpec((1, 1, BLK), lambda i: (i, 0, 0)),
        ],
        out_specs=pl.BlockSpec((1, 1, BLK), lambda i: (i, 0, 0)),
        out_shape=jax.ShapeDtypeStruct((NBLK, 1, BLK), jnp.float32),
    )(image, W_img, b_img.reshape(1, F), uf, itf, ub3, ib3)
    return out.reshape(B)


def kernel(image, user, item, user_factors, item_factors, user_biases,
           item_biases, W_img, b_img):
    user = user.astype(jnp.int32)
    item = item.astype(jnp.int32)
    uf, itf, ub, ib = _sc_gather(
        user, item, user_factors.T, item_factors.T,
        user_biases.T, item_biases.T)
    ub3 = ub.reshape(B // 512, 1, 512)
    ib3 = ib.reshape(B // 512, 1, 512)
    return _tc_combine(image, W_img, b_img, uf, itf, ub3, ib3)


# speedup vs baseline: 4.9744x; 1.0818x over previous
import functools

import jax
import jax.numpy as jnp
from jax import lax
from jax.experimental import pallas as pl
from jax.experimental.pallas import tpu as pltpu
from jax.experimental.pallas import tpu_sc as plsc

B = 4096
F = 64
IMG_DIM = 512
NC, NS = 2, 16
NW = NC * NS
BPW = B // NW
NBUF = 6


def _sc_gather(user, item, uf_t, if_t, ub_t, ib_t):
    mesh = plsc.VectorSubcoreMesh(
        core_axis_name="c", subcore_axis_name="s", num_cores=NC, num_subcores=NS
    )

    @functools.partial(
        pl.kernel,
        out_type=(
            jax.ShapeDtypeStruct((B, F), jnp.float32),
            jax.ShapeDtypeStruct((B,), jnp.float32),
            jax.ShapeDtypeStruct((B,), jnp.float32),
            jax.ShapeDtypeStruct((B,), jnp.float32),
        ),
        mesh=mesh,
        scratch_types=[
            pltpu.VMEM((BPW,), jnp.int32),
            pltpu.VMEM((BPW,), jnp.int32),
            pltpu.VMEM((NBUF, F, 128), jnp.float32),
            pltpu.VMEM((NBUF, F, 128), jnp.float32),
            pltpu.VMEM((BPW, F), jnp.float32),
            pltpu.VMEM((NBUF, 128), jnp.float32),
            pltpu.VMEM((NBUF, 128), jnp.float32),
            pltpu.VMEM((BPW,), jnp.float32),
            pltpu.VMEM((BPW,), jnp.float32),
            pltpu.VMEM((BPW,), jnp.float32),
        ] + [pltpu.SemaphoreType.DMA] * NBUF,
        compiler_params=pltpu.CompilerParams(needs_layout_passes=False),
    )
    def k(user_hbm, item_hbm, uft_hbm, ift_hbm, ubt_hbm, ibt_hbm,
          uf_out, itdot_out, ub_out, ib_out,
          uidx_v, iidx_v, ublocks_v, iblocks_v, uf_v, ubw_v, ibw_v,
          itdot_v, ubsel_v, ibsel_v, *sems):
        wid = lax.axis_index("s") * NC + lax.axis_index("c")
        base = wid * BPW
        pltpu.sync_copy(user_hbm.at[pl.ds(base, BPW)], uidx_v)
        pltpu.sync_copy(item_hbm.at[pl.ds(base, BPW)], iidx_v)
        lane = lax.iota(jnp.int32, 16)

        def idx_at(ref, j):
            chunk = ref[pl.ds((j // 16) * 16, 16)]
            return jnp.sum(jnp.where(lane == j % 16, chunk, 0))

        def start_fetch(j):
            su = idx_at(uidx_v, j)
            si = idx_at(iidx_v, j)
            t0u = pl.multiple_of((su >> 7) << 7, 128)
            t0i = pl.multiple_of((si >> 7) << 7, 128)
            sem = sems[j % NBUF]
            pltpu.async_copy(
                uft_hbm.at[:, pl.ds(t0u, 128)], ublocks_v.at[j % NBUF], sem)
            pltpu.async_copy(
                ift_hbm.at[:, pl.ds(t0i, 128)], iblocks_v.at[j % NBUF], sem)
            pltpu.async_copy(
                ubt_hbm.at[:, pl.ds(t0u, 128)], ubw_v.at[pl.ds(j % NBUF, 1)], sem)
            pltpu.async_copy(
                ibt_hbm.at[:, pl.ds(t0i, 128)], ibw_v.at[pl.ds(j % NBUF, 1)], sem)

        def wait_fetch(j):
            sem = sems[j % NBUF]
            pltpu.make_async_copy(
                uft_hbm.at[:, pl.ds(0, 128)], ublocks_v.at[j % NBUF], sem).wait()
            pltpu.make_async_copy(
                ift_hbm.at[:, pl.ds(0, 128)], iblocks_v.at[j % NBUF], sem).wait()
            pltpu.make_async_copy(
                ubt_hbm.at[:, pl.ds(0, 128)], ubw_v.at[pl.ds(j % NBUF, 1)], sem).wait()
            pltpu.make_async_copy(
                ibt_hbm.at[:, pl.ds(0, 128)], ibw_v.at[pl.ds(j % NBUF, 1)], sem).wait()

        for j in range(NBUF):
            start_fetch(j)
        for g in range(BPW // 16):
            uchunk = uidx_v[pl.ds(g * 16, 16)]
            ichunk = iidx_v[pl.ds(g * 16, 16)]
            ubacc = jnp.zeros((16,), jnp.float32)
            ibacc = jnp.zeros((16,), jnp.float32)
            dotacc = jnp.zeros((16,), jnp.float32)
            for l in range(16):
                j = g * 16 + l
                cu = jnp.sum(jnp.where(lane == l, uchunk, 0)) & 127
                ci = jnp.sum(jnp.where(lane == l, ichunk, 0)) & 127
                wait_fetch(j)
                prodacc = jnp.zeros((16,), jnp.float32)
                for gg in range(F // 16):
                    rows = gg * 16 + lane
                    uvals = plsc.load_gather(
                        ublocks_v.at[j % NBUF],
                        [rows, jnp.broadcast_to(cu, (16,))])
                    uf_v[j, pl.ds(gg * 16, 16)] = uvals
                    ivals = plsc.load_gather(
                        iblocks_v.at[j % NBUF],
                        [rows, jnp.broadcast_to(ci, (16,))])
                    prodacc = prodacc + uvals * ivals
                dotacc = jnp.where(lane == l, jnp.sum(prodacc), dotacc)
                ubval = plsc.load_gather(
                    ubw_v, [jnp.broadcast_to(jnp.int32(j % NBUF), (16,)),
                            jnp.broadcast_to(cu, (16,))])
                ibval = plsc.load_gather(
                    ibw_v, [jnp.broadcast_to(jnp.int32(j % NBUF), (16,)),
                            jnp.broadcast_to(ci, (16,))])
                ubacc = jnp.where(lane == l, ubval, ubacc)
                ibacc = jnp.where(lane == l, ibval, ibacc)
                if j + NBUF < BPW:
                    start_fetch(j + NBUF)
            sl = pl.ds(g * 16, 16)
            ubsel_v[sl] = ubacc
            ibsel_v[sl] = ibacc
            itdot_v[sl] = dotacc
        pltpu.sync_copy(uf_v, uf_out.at[pl.ds(base, BPW)])
        pltpu.sync_copy(itdot_v, itdot_out.at[pl.ds(base, BPW)])
        pltpu.sync_copy(ubsel_v, ub_out.at[pl.ds(base, BPW)])
        pltpu.sync_copy(ibsel_v, ib_out.at[pl.ds(base, BPW)])

    return k(user, item, uf_t, if_t, ub_t, ib_t)


def _tc_combine(image, W_img, b_img, uf, itdot3, ub3, ib3):
    BLK = 512
    NBLK = B // BLK

    def body(img_ref, w_ref, b_ref, uf_ref, itd_ref, ub_ref, ib_ref, out_ref):
        imf = jnp.dot(img_ref[...], w_ref[...],
                      preferred_element_type=jnp.float32) + b_ref[...]
        pred = (ub_ref[0, 0, :] + ib_ref[0, 0, :] + 0.5 * itd_ref[0, 0, :]
                + jnp.sum(uf_ref[...] * (0.5 * imf), axis=1))
        out_ref[0, 0, :] = pred

    out = pl.pallas_call(
        body,
        grid=(NBLK,),
        in_specs=[
            pl.BlockSpec((BLK, IMG_DIM), lambda i: (i, 0)),
            pl.BlockSpec((IMG_DIM, F), lambda i: (0, 0)),
            pl.BlockSpec((1, F), lambda i: (0, 0)),
            pl.BlockSpec((BLK, F), lambda i: (i, 0)),
            pl.BlockSpec((1, 1, BLK), lambda i: (i, 0, 0)),
            pl.BlockSpec((1, 1, BLK), lambda i: (i, 0, 0)),
            pl.BlockSpec((1, 1, BLK), lambda i: (i, 0, 0)),
        ],
        out_specs=pl.BlockSpec((1, 1, BLK), lambda i: (i, 0, 0)),
        out_shape=jax.ShapeDtypeStruct((NBLK, 1, BLK), jnp.float32),
    )(image, W_img, b_img.reshape(1, F), uf, itdot3, ub3, ib3)
    return out.reshape(B)


def kernel(image, user, item, user_factors, item_factors, user_biases,
           item_biases, W_img, b_img):
    user = user.astype(jnp.int32)
    item = item.astype(jnp.int32)
    uf, itdot, ub, ib = _sc_gather(
        user, item, user_factors.T, item_factors.T,
        user_biases.T, item_biases.T)
    itdot3 = itdot.reshape(B // 512, 1, 512)
    ub3 = ub.reshape(B // 512, 1, 512)
    ib3 = ib.reshape(B // 512, 1, 512)
    return _tc_combine(image, W_img, b_img, uf, itdot3, ub3, ib3)
